# baseline (speedup 1.0000x reference)
import jax
import jax.numpy as jnp
from jax import lax
from jax.experimental import pallas as pl
from jax.experimental.pallas import tpu as pltpu
from jax.experimental.pallas import tpu_sc as plsc

_K = 512
_N = 8192
_R = 64
_L = 16
_NB = 256
_NBLK = _N // _L


def _sc_body(x_hbm, out_hbm, row_v, keys_v, cand_a, cand_b, hist_v, cum_v):
    lanes16 = lax.iota(jnp.int32, _L)
    lanebase = lanes16 * _NB
    ones16 = jnp.ones((_L,), jnp.int32)
    zeros16 = jnp.zeros((_L,), jnp.int32)
    wid = lax.axis_index("s") * 2 + lax.axis_index("c")

    cum_v[pl.ds(_NB, _L)] = zeros16

    def zero_hist():
        def z(i, carry):
            hist_v[pl.ds(i * _L, _L)] = zeros16
            return carry

        lax.fori_loop(0, _L * _NB // _L, z, jnp.int32(0), unroll=8)

    def select_bucket(krem):
        def merge_body(gi, carry):
            g = _NB // _L - 1 - gi
            tot = zeros16
            for l in range(_L):
                tot = tot + hist_v[pl.ds(l * _NB + g * _L, _L)]
            rcs = lax.rev(plsc.cumsum(lax.rev(tot, (0,))), (0,))
            cum_v[pl.ds(g * _L, _L)] = rcs + carry
            return carry + jnp.broadcast_to(jnp.sum(tot), (_L,))

        lax.fori_loop(0, _NB // _L, merge_body, zeros16)

        def nge_body(g, nge):
            cum_g = cum_v[pl.ds(g * _L, _L)]
            return nge + plsc.all_reduce_population_count(cum_g >= krem)

        nge = lax.fori_loop(0, _NB // _L, nge_body, zeros16, unroll=4)
        bsel = nge - 1
        cnt_gt = plsc.load_gather(cum_v, [bsel + 1])
        return bsel, krem - cnt_gt

    for r in range(2):
        row = wid * 2 + r
        pltpu.sync_copy(x_hbm.at[pl.ds(row * _N, _N)], row_v)

        zero_hist()

        def l1(i, carry):
            sl = pl.ds(i * _L, _L)
            b = plsc.bitcast(row_v[sl], jnp.int32)
            k = b ^ ((b >> 31) & jnp.int32(0x7FFFFFFF))
            keys_v[sl] = k
            digit = ((k >> 24) & 255) ^ 128
            plsc.addupdate_scatter(hist_v, [lanebase + digit], ones16)
            return carry

        lax.fori_loop(0, _NBLK, l1, jnp.int32(0), unroll=4)

        krem = jnp.full((_L,), _K, jnp.int32)
        b1, krem = select_bucket(krem)

        def c1(i, cnt):
            k = keys_v[pl.ds(i * _L, _L)]
            sel = (((k >> 24) & 255) ^ 128) == b1
            plsc.store_compressed(cand_a.at[pl.ds(cnt, _L)], k, mask=sel)
            return cnt + jnp.sum(sel.astype(jnp.int32))

        m1 = lax.fori_loop(0, _NBLK, c1, jnp.int32(0), unroll=2)

        zero_hist()
        nb2 = (m1 + _L - 1) // _L

        def h2(i, carry):
            base = i * _L
            k = cand_a[pl.ds(base, _L)]
            valid = (base + lanes16) < m1
            digit = (k >> 16) & 255
            plsc.addupdate_scatter(
                hist_v, [lanebase + digit], ones16, mask=valid)
            return carry

        lax.fori_loop(0, nb2, h2, jnp.int32(0))
        b2, krem = select_bucket(krem)

        def c2(i, cnt):
            base = i * _L
            k = cand_a[pl.ds(base, _L)]
            sel = (((k >> 16) & 255) == b2) & ((base + lanes16) < m1)
            plsc.store_compressed(cand_b.at[pl.ds(cnt, _L)], k, mask=sel)
            return cnt + jnp.sum(sel.astype(jnp.int32))

        m2 = lax.fori_loop(0, nb2, c2, jnp.int32(0))

        nb3 = (m2 + _L - 1) // _L
        prefix = zeros16
        for bit in range(15, -1, -1):
            cand = prefix | (1 << bit)

            def cb(i, acc, cand=cand):
                base = i * _L
                k = cand_b[pl.ds(base, _L)]
                sel = ((k & 0xFFFF) >= cand) & ((base + lanes16) < m2)
                return acc + plsc.all_reduce_population_count(sel)

            cnt = lax.fori_loop(0, nb3, cb, zeros16)
            prefix = jnp.where(cnt >= krem, cand, prefix)

        thr = (((b1 ^ 128) & 255) << 24) | (b2 << 16) | prefix

        def outp(i, carry):
            sl = pl.ds(i * _L, _L)
            row_v[sl] = jnp.where(keys_v[sl] >= thr, row_v[sl],
                                  jnp.float32(0.0))
            return carry

        lax.fori_loop(0, _NBLK, outp, jnp.int32(0), unroll=8)
        pltpu.sync_copy(row_v, out_hbm.at[pl.ds(row * _N, _N)])


def kernel(x):
    mesh = plsc.VectorSubcoreMesh(core_axis_name="c", subcore_axis_name="s")
    flat = x.reshape(-1)
    out = pl.kernel(
        _sc_body,
        out_type=jax.ShapeDtypeStruct((_R * _N,), jnp.float32),
        mesh=mesh,
        compiler_params=pltpu.CompilerParams(needs_layout_passes=False),
        scratch_types=[
            pltpu.VMEM((_N,), jnp.float32),
            pltpu.VMEM((_N,), jnp.int32),
            pltpu.VMEM((_N + _L,), jnp.int32),
            pltpu.VMEM((_N + _L,), jnp.int32),
            pltpu.VMEM((_L * _NB,), jnp.int32),
            pltpu.VMEM((_NB + _L,), jnp.int32),
        ],
    )(flat)
    return out.reshape(x.shape)


# speedup vs baseline: 1.6481x; 1.6481x over previous
import jax
import jax.numpy as jnp
from jax import lax
from jax.experimental import pallas as pl
from jax.experimental.pallas import tpu as pltpu
from jax.experimental.pallas import tpu_sc as plsc

_K = 512
_N = 8192
_R = 64
_L = 16
_NB = 256
_NBLK = _N // _L


def _sc_body(x_hbm, out_hbm, row0_v, row1_v, cand_a, cand_b, hist_v, cum_v,
             sem_i0, sem_i1, sem_o0, sem_o1):
    lanes16 = lax.iota(jnp.int32, _L)
    lanebase = lanes16 * _NB
    ones16 = jnp.ones((_L,), jnp.int32)
    zeros16 = jnp.zeros((_L,), jnp.int32)
    wid = lax.axis_index("s") * 2 + lax.axis_index("c")

    cum_v[pl.ds(_NB, _L)] = zeros16

    def zero_hist():
        @plsc.parallel_loop(0, _L * _NB // _L, unroll=8)
        def _z(i):
            hist_v[pl.ds(i * _L, _L)] = zeros16

    def select_bucket(krem):
        def merge_body(gi, carry):
            g = _NB // _L - 1 - gi
            tot = zeros16
            for l in range(_L):
                tot = tot + hist_v[pl.ds(l * _NB + g * _L, _L)]
            rcs = lax.rev(plsc.cumsum(lax.rev(tot, (0,))), (0,))
            cum_v[pl.ds(g * _L, _L)] = rcs + carry
            return carry + jnp.broadcast_to(jnp.sum(tot), (_L,))

        lax.fori_loop(0, _NB // _L, merge_body, zeros16)

        def nge_body(g, nge):
            cum_g = cum_v[pl.ds(g * _L, _L)]
            return nge + plsc.all_reduce_population_count(cum_g >= krem)

        nge = lax.fori_loop(0, _NB // _L, nge_body, zeros16, unroll=4)
        bsel = nge - 1
        cnt_gt = plsc.load_gather(cum_v, [bsel + 1])
        return bsel, krem - cnt_gt

    def top_digit(b):
        return (lax.shift_right_logical(b, 24) ^ 128) ^ ((b >> 31) & 127)

    def process_row(row_v):
        zero_hist()

        @plsc.parallel_loop(0, _NBLK, unroll=4)
        def _l1(i):
            d1 = top_digit(row_v[pl.ds(i * _L, _L)])
            plsc.addupdate_scatter(hist_v, [lanebase | d1], ones16)

        krem = jnp.full((_L,), _K, jnp.int32)
        b1, krem = select_bucket(krem)
        sgn_neg = b1 < 128
        flip8 = jnp.where(sgn_neg, jnp.full((_L,), 255, jnp.int32), zeros16)
        flip16 = jnp.where(sgn_neg, jnp.full((_L,), 0xFFFF, jnp.int32),
                           zeros16)

        @plsc.parallel_loop(0, _NBLK, unroll=2, carry=zeros16)
        def c1(i, cnt):
            b = row_v[pl.ds(i * _L, _L)]
            sel = top_digit(b) == b1
            seli = sel.astype(jnp.int32)
            pos = (plsc.cumsum(seli) - seli) + cnt
            plsc.store_scatter(cand_a, [pos], b, mask=sel)
            return cnt + plsc.all_reduce_population_count(sel)

        m1 = jnp.max(c1)
        nb2 = (m1 + _L - 1) // _L
        m1s = c1

        zero_hist()

        @plsc.parallel_loop(0, nb2, carry=None)
        def _h2(i):
            base = i * _L
            b = cand_a[pl.ds(base, _L)]
            d2 = (lax.shift_right_logical(b, 16) & 255) ^ flip8
            valid = (base + lanes16) < m1s
            plsc.addupdate_scatter(hist_v, [lanebase | d2], ones16,
                                   mask=valid)

        b2, krem = select_bucket(krem)

        @plsc.parallel_loop(0, nb2, carry=zeros16)
        def c2(i, cnt):
            base = i * _L
            b = cand_a[pl.ds(base, _L)]
            d2 = (lax.shift_right_logical(b, 16) & 255) ^ flip8
            sel = (d2 == b2) & ((base + lanes16) < m1s)
            seli = sel.astype(jnp.int32)
            pos = (plsc.cumsum(seli) - seli) + cnt
            plsc.store_scatter(cand_b, [pos], b, mask=sel)
            return cnt + plsc.all_reduce_population_count(sel)

        m2 = jnp.max(c2)
        nb3 = (m2 + _L - 1) // _L
        m2s = c2

        prefix = zeros16
        for bit in range(15, -1, -1):
            cand = prefix | (1 << bit)

            @plsc.parallel_loop(0, nb3, carry=zeros16)
            def cb(i, acc, cand=cand):
                base = i * _L
                b = cand_b[pl.ds(base, _L)]
                low = (b & 0xFFFF) ^ flip16
                sel = (low >= cand) & ((base + lanes16) < m2s)
                return acc + plsc.all_reduce_population_count(sel)

            prefix = jnp.where(cb >= krem, cand, prefix)

        top = jnp.where(sgn_neg, (b1 ^ 128) ^ 127, b1 ^ 128)
        thr_bits = (top << 24) | ((b2 ^ flip8) << 16) | (prefix ^ flip16)
        thr_f = plsc.bitcast(thr_bits, jnp.float32)

        @plsc.parallel_loop(0, _NBLK, unroll=4)
        def _outp(i):
            sl = pl.ds(i * _L, _L)
            b = row_v[sl]
            keep = plsc.bitcast(b, jnp.float32) >= thr_f
            row_v[sl] = jnp.where(keep, b, zeros16)

    row0 = wid * 2
    row1 = row0 + 1
    cin0 = pltpu.async_copy(x_hbm.at[pl.ds(row0 * _N, _N)], row0_v, sem_i0)
    cin1 = pltpu.async_copy(x_hbm.at[pl.ds(row1 * _N, _N)], row1_v, sem_i1)
    cin0.wait()
    process_row(row0_v)
    cout0 = pltpu.async_copy(row0_v, out_hbm.at[pl.ds(row0 * _N, _N)], sem_o0)
    cin1.wait()
    process_row(row1_v)
    cout1 = pltpu.async_copy(row1_v, out_hbm.at[pl.ds(row1 * _N, _N)], sem_o1)
    cout0.wait()
    cout1.wait()


def kernel(x):
    mesh = plsc.VectorSubcoreMesh(core_axis_name="c", subcore_axis_name="s")
    bits = jax.lax.bitcast_convert_type(x, jnp.int32).reshape(-1)
    out = pl.kernel(
        _sc_body,
        out_type=jax.ShapeDtypeStruct((_R * _N,), jnp.int32),
        mesh=mesh,
        compiler_params=pltpu.CompilerParams(needs_layout_passes=False),
        scratch_types=[
            pltpu.VMEM((_N,), jnp.int32),
            pltpu.VMEM((_N,), jnp.int32),
            pltpu.VMEM((_N + _L,), jnp.int32),
            pltpu.VMEM((_N + _L,), jnp.int32),
            pltpu.VMEM((_L * _NB,), jnp.int32),
            pltpu.VMEM((_NB + _L,), jnp.int32),
            pltpu.SemaphoreType.DMA,
            pltpu.SemaphoreType.DMA,
            pltpu.SemaphoreType.DMA,
            pltpu.SemaphoreType.DMA,
        ],
    )(bits)
    return jax.lax.bitcast_convert_type(out.reshape(x.shape), jnp.float32)


# speedup vs baseline: 1.8222x; 1.1056x over previous
import jax
import jax.numpy as jnp
from jax import lax
from jax.experimental import pallas as pl
from jax.experimental.pallas import tpu as pltpu
from jax.experimental.pallas import tpu_sc as plsc

_K = 512
_N = 8192
_R = 64
_L = 16
_NB = 256
_NBLK = _N // _L


def _sc_body(x_hbm, out_hbm, row0_v, row1_v, cand_a, cand_b, hist_v, cum_v,
             scr_v, sem_i0, sem_i1, sem_o0, sem_o1):
    lanes16 = lax.iota(jnp.int32, _L)
    ones16 = jnp.ones((_L,), jnp.int32)
    zeros16 = jnp.zeros((_L,), jnp.int32)
    wid = lax.axis_index("s") * 2 + lax.axis_index("c")

    cum_v[pl.ds(_NB, _L)] = zeros16

    def zero_hist():
        @plsc.parallel_loop(0, _NB // _L, unroll=4)
        def _z(i):
            hist_v[pl.ds(i * _L, _L)] = zeros16

    def select_bucket(krem):
        def merge_body(gi, carry):
            g = _NB // _L - 1 - gi
            tot = hist_v[pl.ds(g * _L, _L)]
            rcs = lax.rev(plsc.cumsum(lax.rev(tot, (0,))), (0,))
            cum_g = rcs + carry
            cum_v[pl.ds(g * _L, _L)] = cum_g
            return jnp.broadcast_to(cum_g[0], (_L,))

        lax.fori_loop(0, _NB // _L, merge_body, zeros16)

        def nge_body(g, nge):
            cum_g = cum_v[pl.ds(g * _L, _L)]
            return nge + plsc.all_reduce_population_count(cum_g >= krem)

        nge = lax.fori_loop(0, _NB // _L, nge_body, zeros16, unroll=4)
        bsel = nge - 1
        cnt_gt = plsc.load_gather(cum_v, [bsel + 1])
        return bsel, krem - cnt_gt

    def top_digit(b):
        return (lax.shift_right_logical(b, 24) ^ 128) ^ ((b >> 31) & 127)

    def process_row(row_v):
        zero_hist()

        @plsc.parallel_loop(0, _NBLK, unroll=4)
        def _l1(i):
            d1 = top_digit(row_v[pl.ds(i * _L, _L)])
            cnt, lastm = plsc.scan_count(d1)
            plsc.addupdate_scatter(hist_v, [d1], cnt, mask=lastm)

        krem = jnp.full((_L,), _K, jnp.int32)
        b1, krem = select_bucket(krem)
        sgn_neg = b1 < 128
        flip8 = jnp.where(sgn_neg, jnp.full((_L,), 255, jnp.int32), zeros16)
        flip16 = jnp.where(sgn_neg, jnp.full((_L,), 0xFFFF, jnp.int32),
                           zeros16)

        @plsc.parallel_loop(0, _NBLK, unroll=4, carry=zeros16)
        def c1(i, cnt):
            b = row_v[pl.ds(i * _L, _L)]
            sel = top_digit(b) == b1
            seli = sel.astype(jnp.int32)
            pos = (plsc.cumsum(seli) - seli) + cnt
            plsc.store_scatter(cand_a, [pos], b, mask=sel)
            return cnt + plsc.all_reduce_population_count(sel)

        m1 = jnp.max(c1)
        nb2 = (m1 + _L - 1) // _L
        m1s = c1

        zero_hist()

        @plsc.parallel_loop(0, nb2, carry=None)
        def _h2(i):
            base = i * _L
            b = cand_a[pl.ds(base, _L)]
            d2 = (lax.shift_right_logical(b, 16) & 255) ^ flip8
            valid = (base + lanes16) < m1s
            cnt, lastm = plsc.scan_count(d2, valid)
            plsc.addupdate_scatter(hist_v, [d2], cnt, mask=lastm & valid)

        b2, krem = select_bucket(krem)

        @plsc.parallel_loop(0, nb2, carry=zeros16)
        def c2(i, cnt):
            base = i * _L
            b = cand_a[pl.ds(base, _L)]
            d2 = (lax.shift_right_logical(b, 16) & 255) ^ flip8
            sel = (d2 == b2) & ((base + lanes16) < m1s)
            seli = sel.astype(jnp.int32)
            pos = (plsc.cumsum(seli) - seli) + cnt
            plsc.store_scatter(cand_b, [pos], b, mask=sel)
            return cnt + plsc.all_reduce_population_count(sel)

        m2 = jnp.max(c2)
        m2s = c2

        def small_case(_):
            b = cand_b[pl.ds(0, _L)]
            low = (b & 0xFFFF) ^ flip16
            lowm = jnp.where(lanes16 < m2s, low,
                             jnp.full((_L,), -1, jnp.int32))
            sk, _sv = plsc.sort_key_val(lowm, lowm, descending=True)
            scr_v[...] = sk
            return plsc.load_gather(scr_v, [krem - 1])

        def big_case(_):
            nb3 = (m2 + _L - 1) // _L
            prefix = zeros16
            for bit in range(15, -1, -1):
                cand = prefix | (1 << bit)

                @plsc.parallel_loop(0, nb3, carry=zeros16)
                def cb(i, acc, cand=cand):
                    base = i * _L
                    b = cand_b[pl.ds(base, _L)]
                    low = (b & 0xFFFF) ^ flip16
                    sel = (low >= cand) & ((base + lanes16) < m2s)
                    return acc + plsc.all_reduce_population_count(sel)

                prefix = jnp.where(cb >= krem, cand, prefix)
            return prefix

        prefix = lax.cond(m2 <= _L, small_case, big_case, 0)

        top = jnp.where(sgn_neg, (b1 ^ 128) ^ 127, b1 ^ 128)
        thr_bits = (top << 24) | ((b2 ^ flip8) << 16) | (prefix ^ flip16)
        thr_f = plsc.bitcast(thr_bits, jnp.float32)

        @plsc.parallel_loop(0, _NBLK, unroll=4)
        def _outp(i):
            sl = pl.ds(i * _L, _L)
            b = row_v[sl]
            keep = plsc.bitcast(b, jnp.float32) >= thr_f
            row_v[sl] = jnp.where(keep, b, zeros16)

    row0 = wid * 2
    row1 = row0 + 1
    cin0 = pltpu.async_copy(x_hbm.at[pl.ds(row0 * _N, _N)], row0_v, sem_i0)
    cin1 = pltpu.async_copy(x_hbm.at[pl.ds(row1 * _N, _N)], row1_v, sem_i1)
    cin0.wait()
    process_row(row0_v)
    cout0 = pltpu.async_copy(row0_v, out_hbm.at[pl.ds(row0 * _N, _N)], sem_o0)
    cin1.wait()
    process_row(row1_v)
    cout1 = pltpu.async_copy(row1_v, out_hbm.at[pl.ds(row1 * _N, _N)], sem_o1)
    cout0.wait()
    cout1.wait()


def kernel(x):
    mesh = plsc.VectorSubcoreMesh(core_axis_name="c", subcore_axis_name="s")
    bits = jax.lax.bitcast_convert_type(x, jnp.int32).reshape(-1)
    out = pl.kernel(
        _sc_body,
        out_type=jax.ShapeDtypeStruct((_R * _N,), jnp.int32),
        mesh=mesh,
        compiler_params=pltpu.CompilerParams(needs_layout_passes=False),
        scratch_types=[
            pltpu.VMEM((_N,), jnp.int32),
            pltpu.VMEM((_N,), jnp.int32),
            pltpu.VMEM((_N + _L,), jnp.int32),
            pltpu.VMEM((_N + _L,), jnp.int32),
            pltpu.VMEM((_NB,), jnp.int32),
            pltpu.VMEM((_NB + _L,), jnp.int32),
            pltpu.VMEM((_L,), jnp.int32),
            pltpu.SemaphoreType.DMA,
            pltpu.SemaphoreType.DMA,
            pltpu.SemaphoreType.DMA,
            pltpu.SemaphoreType.DMA,
        ],
    )(bits)
    return jax.lax.bitcast_convert_type(out.reshape(x.shape), jnp.float32)


# speedup vs baseline: 1.8655x; 1.0237x over previous
import jax
import jax.numpy as jnp
from jax import lax
from jax.experimental import pallas as pl
from jax.experimental.pallas import tpu as pltpu
from jax.experimental.pallas import tpu_sc as plsc

_K = 512
_N = 8192
_R = 64
_L = 16
_NB = 256
_NBLK = _N // _L


def _sc_body(x_hbm, out_hbm, row0_v, row1_v, cand_a, cand_b, hist_v, cum_v,
             scr_v, dig_v, sem_i0, sem_i1, sem_o0, sem_o1):
    lanes16 = lax.iota(jnp.int32, _L)
    ones16 = jnp.ones((_L,), jnp.int32)
    zeros16 = jnp.zeros((_L,), jnp.int32)
    wid = lax.axis_index("s") * 2 + lax.axis_index("c")

    cum_v[pl.ds(_NB, _L)] = zeros16

    def zero_hist():
        @plsc.parallel_loop(0, _NB // _L, unroll=4)
        def _z(i):
            hist_v[pl.ds(i * _L, _L)] = zeros16

    def select_bucket(krem):
        def merge_body(gi, carry):
            g = _NB // _L - 1 - gi
            tot = hist_v[pl.ds(g * _L, _L)]
            rcs = lax.rev(plsc.cumsum(lax.rev(tot, (0,))), (0,))
            cum_g = rcs + carry
            cum_v[pl.ds(g * _L, _L)] = cum_g
            return jnp.broadcast_to(cum_g[0], (_L,))

        lax.fori_loop(0, _NB // _L, merge_body, zeros16)

        def nge_body(g, nge):
            cum_g = cum_v[pl.ds(g * _L, _L)]
            return nge + plsc.all_reduce_population_count(cum_g >= krem)

        nge = lax.fori_loop(0, _NB // _L, nge_body, zeros16, unroll=4)
        bsel = nge - 1
        cnt_gt = plsc.load_gather(cum_v, [bsel + 1])
        return bsel, krem - cnt_gt

    def top_digit(b):
        return (lax.shift_right_logical(b, 24) ^ 128) ^ ((b >> 31) & 127)

    def process_row(row_v):
        zero_hist()

        @plsc.parallel_loop(0, _NBLK, unroll=4)
        def _l1(i):
            d1 = top_digit(row_v[pl.ds(i * _L, _L)])
            dig_v[pl.ds(i * _L, _L)] = d1
            cnt, lastm = plsc.scan_count(d1)
            plsc.addupdate_scatter(hist_v, [d1], cnt, mask=lastm)

        krem = jnp.full((_L,), _K, jnp.int32)
        b1, krem = select_bucket(krem)
        sgn_neg = b1 < 128
        flip8 = jnp.where(sgn_neg, jnp.full((_L,), 255, jnp.int32), zeros16)
        flip16 = jnp.where(sgn_neg, jnp.full((_L,), 0xFFFF, jnp.int32),
                           zeros16)

        @plsc.parallel_loop(0, _NBLK, unroll=8, carry=zeros16)
        def c1(i, cnt):
            b = row_v[pl.ds(i * _L, _L)]
            sel = dig_v[pl.ds(i * _L, _L)] == b1
            seli = sel.astype(jnp.int32)
            pos = (plsc.cumsum(seli) - seli) + cnt
            plsc.store_scatter(cand_a, [pos], b, mask=sel)
            return cnt + plsc.all_reduce_population_count(sel)

        m1 = jnp.max(c1)
        nb2 = (m1 + _L - 1) // _L
        m1s = c1

        zero_hist()

        @plsc.parallel_loop(0, nb2, carry=None)
        def _h2(i):
            base = i * _L
            b = cand_a[pl.ds(base, _L)]
            d2 = (lax.shift_right_logical(b, 16) & 255) ^ flip8
            valid = (base + lanes16) < m1s
            cnt, lastm = plsc.scan_count(d2, valid)
            plsc.addupdate_scatter(hist_v, [d2], cnt, mask=lastm & valid)

        b2, krem = select_bucket(krem)

        @plsc.parallel_loop(0, nb2, carry=zeros16)
        def c2(i, cnt):
            base = i * _L
            b = cand_a[pl.ds(base, _L)]
            d2 = (lax.shift_right_logical(b, 16) & 255) ^ flip8
            sel = (d2 == b2) & ((base + lanes16) < m1s)
            seli = sel.astype(jnp.int32)
            pos = (plsc.cumsum(seli) - seli) + cnt
            plsc.store_scatter(cand_b, [pos], b, mask=sel)
            return cnt + plsc.all_reduce_population_count(sel)

        m2 = jnp.max(c2)
        m2s = c2

        def small_case(_):
            b = cand_b[pl.ds(0, _L)]
            low = (b & 0xFFFF) ^ flip16
            lowm = jnp.where(lanes16 < m2s, low,
                             jnp.full((_L,), -1, jnp.int32))
            sk, _sv = plsc.sort_key_val(lowm, lowm, descending=True)
            scr_v[...] = sk
            return plsc.load_gather(scr_v, [krem - 1])

        def big_case(_):
            nb3 = (m2 + _L - 1) // _L
            prefix = zeros16
            for bit in range(15, -1, -1):
                cand = prefix | (1 << bit)

                @plsc.parallel_loop(0, nb3, carry=zeros16)
                def cb(i, acc, cand=cand):
                    base = i * _L
                    b = cand_b[pl.ds(base, _L)]
                    low = (b & 0xFFFF) ^ flip16
                    sel = (low >= cand) & ((base + lanes16) < m2s)
                    return acc + plsc.all_reduce_population_count(sel)

                prefix = jnp.where(cb >= krem, cand, prefix)
            return prefix

        prefix = lax.cond(m2 <= _L, small_case, big_case, 0)

        top = jnp.where(sgn_neg, (b1 ^ 128) ^ 127, b1 ^ 128)
        thr_bits = (top << 24) | ((b2 ^ flip8) << 16) | (prefix ^ flip16)
        thr_f = plsc.bitcast(thr_bits, jnp.float32)

        @plsc.parallel_loop(0, _NBLK, unroll=4)
        def _outp(i):
            sl = pl.ds(i * _L, _L)
            b = row_v[sl]
            keep = plsc.bitcast(b, jnp.float32) >= thr_f
            row_v[sl] = jnp.where(keep, b, zeros16)

    row0 = wid * 2
    row1 = row0 + 1
    cin0 = pltpu.async_copy(x_hbm.at[pl.ds(row0 * _N, _N)], row0_v, sem_i0)
    cin1 = pltpu.async_copy(x_hbm.at[pl.ds(row1 * _N, _N)], row1_v, sem_i1)
    cin0.wait()
    process_row(row0_v)
    cout0 = pltpu.async_copy(row0_v, out_hbm.at[pl.ds(row0 * _N, _N)], sem_o0)
    cin1.wait()
    process_row(row1_v)
    cout1 = pltpu.async_copy(row1_v, out_hbm.at[pl.ds(row1 * _N, _N)], sem_o1)
    cout0.wait()
    cout1.wait()


def kernel(x):
    mesh = plsc.VectorSubcoreMesh(core_axis_name="c", subcore_axis_name="s")
    bits = jax.lax.bitcast_convert_type(x, jnp.int32).reshape(-1)
    out = pl.kernel(
        _sc_body,
        out_type=jax.ShapeDtypeStruct((_R * _N,), jnp.int32),
        mesh=mesh,
        compiler_params=pltpu.CompilerParams(needs_layout_passes=False),
        scratch_types=[
            pltpu.VMEM((_N,), jnp.int32),
            pltpu.VMEM((_N,), jnp.int32),
            pltpu.VMEM((_N + _L,), jnp.int32),
            pltpu.VMEM((_N + _L,), jnp.int32),
            pltpu.VMEM((_NB,), jnp.int32),
            pltpu.VMEM((_NB + _L,), jnp.int32),
            pltpu.VMEM((_L,), jnp.int32),
            pltpu.VMEM((_N,), jnp.int32),
            pltpu.SemaphoreType.DMA,
            pltpu.SemaphoreType.DMA,
            pltpu.SemaphoreType.DMA,
            pltpu.SemaphoreType.DMA,
        ],
    )(bits)
    return jax.lax.bitcast_convert_type(out.reshape(x.shape), jnp.float32)
